# Initial kernel scaffold; baseline (speedup 1.0000x reference)
import jax
import jax.numpy as jnp
from jax.experimental import pallas as pl


def kernel(xyz, params):
    raise NotImplementedError("write your pallas kernel here")


# speedup vs baseline: 1.5431x; 1.5431x over previous
import functools

import jax
import jax.numpy as jnp
from jax.experimental import pallas as pl
from jax.experimental.pallas import tpu as pltpu

_BATCH = 4
_K = 32


def _fps_body(s, n, x_ref, y_ref, z_ref, idx_ref):
    b = x_ref.shape[0]
    x = x_ref[...]
    y = y_ref[...]
    z = z_ref[...]
    lane = jax.lax.broadcasted_iota(jnp.int32, (b, n), 1)

    def step(j, carry):
        dists, far, acc = carry
        mask = lane == far
        cx = jnp.sum(jnp.where(mask, x, 0.0), axis=1, keepdims=True)
        cy = jnp.sum(jnp.where(mask, y, 0.0), axis=1, keepdims=True)
        cz = jnp.sum(jnp.where(mask, z, 0.0), axis=1, keepdims=True)
        d = (x - cx) ** 2 + (y - cy) ** 2 + (z - cz) ** 2
        dists = jnp.minimum(dists, d)
        m = jnp.max(dists, axis=1, keepdims=True)
        nxt = jnp.min(jnp.where(dists == m, lane, n), axis=1, keepdims=True)
        acc = jnp.concatenate([acc[:, 1:], far], axis=1)
        return dists, nxt.astype(jnp.int32), acc

    init = (jnp.full((b, n), 1e10, jnp.float32), jnp.zeros((b, 1), jnp.int32),
            jnp.zeros((b, s), jnp.int32))
    _, _, acc = jax.lax.fori_loop(0, s, step, init)
    idx_ref[...] = acc


def _fps_pallas(pts, s):
    b, n, _ = pts.shape
    x = pts[:, :, 0]
    y = pts[:, :, 1]
    z = pts[:, :, 2]
    return pl.pallas_call(
        functools.partial(_fps_body, s, n),
        out_shape=jax.ShapeDtypeStruct((b, s), jnp.int32),
    )(x, y, z)


def _knn_body(n, k, ctr_ref, px_ref, py_ref, pz_ref, idx_ref):
    cb = ctr_ref.shape[1]
    cx = ctr_ref[0, :, 0:1]
    cy = ctr_ref[0, :, 1:2]
    cz = ctr_ref[0, :, 2:3]
    px = px_ref[0]
    py = py_ref[0]
    pz = pz_ref[0]
    d0 = (cx - px) ** 2 + (cy - py) ** 2 + (cz - pz) ** 2
    lane = jax.lax.broadcasted_iota(jnp.int32, (cb, n), 1)

    def step(i, carry):
        d, acc = carry
        m = jnp.min(d, axis=1, keepdims=True)
        j = jnp.min(jnp.where(d == m, lane, n), axis=1, keepdims=True)
        ji = j.astype(jnp.int32)
        acc = jnp.concatenate([acc[:, 1:], ji], axis=1)
        return jnp.where(lane == j, jnp.float32(3.4e38), d), acc

    _, acc = jax.lax.fori_loop(0, k, step, (d0, jnp.zeros((cb, k), jnp.int32)))
    idx_ref[0] = acc


def _knn_pallas(centers, pts, k):
    b, s, _ = centers.shape
    n = pts.shape[1]
    cb = min(s, 256)
    px = pts[:, None, :, 0]
    py = pts[:, None, :, 1]
    pz = pts[:, None, :, 2]
    return pl.pallas_call(
        functools.partial(_knn_body, n, k),
        grid=(b, s // cb),
        in_specs=[
            pl.BlockSpec((1, cb, 3), lambda bi, si: (bi, si, 0)),
            pl.BlockSpec((1, 1, n), lambda bi, si: (bi, 0, 0)),
            pl.BlockSpec((1, 1, n), lambda bi, si: (bi, 0, 0)),
            pl.BlockSpec((1, 1, n), lambda bi, si: (bi, 0, 0)),
        ],
        out_specs=pl.BlockSpec((1, cb, k), lambda bi, si: (bi, si, 0)),
        out_shape=jax.ShapeDtypeStruct((b, s, k), jnp.int32),
    )(centers, px, py, pz)


def _conv(x, W, b):
    return jnp.einsum('bcn,cd->bdn', x, W) + b[None, :, None]


def _bn(x, g, be):
    m = x.mean(axis=(0, 2), keepdims=True)
    v = x.var(axis=(0, 2), keepdims=True)
    return g[None, :, None] * (x - m) / jnp.sqrt(v + 1e-5) + be[None, :, None]


def _cbr(x, W, b, g, be):
    return jax.nn.relu(_bn(_conv(x, W, b), g, be))


def _res(x, p, pre):
    h = _cbr(x, p[pre + 'W1'], p[pre + 'b1'], p[pre + 'g1'], p[pre + 'be1'])
    h = _bn(_conv(h, p[pre + 'W2'], p[pre + 'b2']), p[pre + 'g2'], p[pre + 'be2'])
    return jax.nn.relu(h + x)


def _gather(x, idx):
    return jnp.take_along_axis(x, idx[..., None], axis=1)


def kernel(xyz, params):
    stages = 4
    k = _K
    embed = 32
    dim_ratio = [2, 2, 2, 2]
    samp_ratio = [2, 2, 2, 2]
    num_b1 = [2, 2, 2, 2]
    num_b2 = [2, 2, 2, 2]

    B = xyz.shape[0]
    n_points = xyz.shape[2]

    f = _cbr(xyz, params['emb_W'], params['emb_b'], params['emb_g'], params['emb_be'])
    f = params['emb_alpha'][None, :, None] * f + params['emb_beta'][None, :, None]
    cur = jnp.transpose(xyz, (0, 2, 1))
    xyz_list = [cur]
    f_list = [f]
    lc = embed
    n = n_points
    for i in range(stages):
        s = n // samp_ratio[i]
        oc = lc * dim_ratio[i]
        f_nc = jnp.transpose(f, (0, 2, 1))
        fidx = _fps_pallas(cur, s)
        xyz_s = _gather(cur, fidx)
        kidx = _knn_pallas(xyz_s, cur, k)
        f_s = _gather(f_nc, fidx)
        xyz_g = _gather(cur, kidx.reshape(B, -1)).reshape(B, s, k, 3)
        f_g = _gather(f_nc, kidx.reshape(B, -1)).reshape(B, s, k, lc)
        grouped = jnp.concatenate([f_g, xyz_g - xyz_s[:, :, None, :]], axis=-1)
        anchor = jnp.concatenate([f_s, jnp.zeros_like(xyz_s)], axis=-1)
        centered = grouped - anchor[:, :, None, :]
        std = jnp.std(centered.reshape(B, -1), axis=-1)[:, None, None, None]
        g = params['s%d_na' % i] * (centered / (std + 1e-5)) + params['s%d_nb' % i]
        x = jnp.transpose(g.reshape(B, s * k, lc + 3), (0, 2, 1))
        x = _cbr(x, params['s%d_tW' % i], params['s%d_tb' % i],
                 params['s%d_tg' % i], params['s%d_tbe' % i])
        for j in range(num_b1[i]):
            x = _res(x, params, 's%d_b1%d_' % (i, j))
        x = jnp.max(x.reshape(B, oc, s, k), axis=-1)
        for j in range(num_b2[i]):
            x = _res(x, params, 's%d_b2%d_' % (i, j))
        f = x
        cur = xyz_s
        lc = oc
        n = s
        xyz_list.append(cur)
        f_list.append(f)
    return tuple(xyz_list), tuple(f_list)
